# baseline (speedup 1.0000x reference)
import math
from functools import partial

import jax
import jax.numpy as jnp
from jax import lax
from jax.experimental import pallas as pl
from jax.experimental.pallas import tpu as pltpu

EPS = 1e-5
GIMG = 32
PB = 128


def _shift_lanes(x, s, zcol):
    if s == 0:
        return x
    if s > 0:
        return jnp.concatenate([x[:, s:], zcol[:, :s]], axis=1)
    return jnp.concatenate([zcol[:, :(-s)], x[:, :s]], axis=1)


def _tin_kernel(x_ref, o_ref):
    o_ref[...] = jnp.transpose(x_ref[...], (2, 0, 1)).astype(jnp.bfloat16)


def _p1_kernel(x_ref, w_ref, b3_ref, y1_ref, st_ref, *, G, W, Cin, Cout):
    xb = x_ref[0]
    rows, hw = xb.shape
    zcol = jnp.zeros((rows, 33), jnp.bfloat16)
    lane = lax.broadcasted_iota(jnp.int32, (1, hw), 1) % W
    zero = jnp.zeros((), jnp.bfloat16)
    shifted = []
    for ky in range(3):
        for kx in range(3):
            s = W * (ky - 1) + (kx - 1)
            t = _shift_lanes(xb, s, zcol)
            if kx == 0:
                t = jnp.where(lane == 0, zero, t)
            elif kx == 2:
                t = jnp.where(lane == W - 1, zero, t)
            shifted.append(t)
    b3c = b3_ref[:, 0:1]
    for i in range(G):
        r0 = i * Cin
        acc = jnp.dot(w_ref[0], shifted[0][r0:r0 + Cin, :],
                      preferred_element_type=jnp.float32)
        for k in range(1, 9):
            acc = acc + jnp.dot(w_ref[k], shifted[k][r0:r0 + Cin, :],
                                preferred_element_type=jnp.float32)
        y = acc + b3c
        y1_ref[0, i * Cout:(i + 1) * Cout, :] = y.astype(jnp.bfloat16)
        s1 = jnp.sum(y, axis=1, keepdims=True)
        s2 = jnp.sum(y * y, axis=1, keepdims=True)
        if i == 0:
            st1, st2 = s1, s2
        else:
            st1, st2 = st1 + s1, st2 + s2
    st_ref[0] = jnp.concatenate([st1, st2], axis=1)


def _p2_kernel(xv_ref, y1_ref, w1_ref, ss_ref, o_ref, *, Cin, Cout, N):
    pb = xv_ref.shape[1]
    xb = xv_ref[...].reshape(Cin, pb * N).astype(jnp.bfloat16)
    y2 = jnp.dot(w1_ref[...], xb,
                 preferred_element_type=jnp.float32).reshape(Cout, pb, N)
    y1c = jnp.transpose(y1_ref[...], (1, 2, 0)).astype(jnp.float32)
    ss = ss_ref[...]
    sc = ss[0][:, 0:1, None]
    sh = ss[1][:, 0:1, None]
    b1c = ss[2][:, 0:1, None]
    y1n = jnp.maximum(y1c * sc + sh, 0.0)
    o_ref[...] = jnp.maximum(y1n + y2 + b1c, 0.0)


@jax.jit
def _forward(x_nchw, w3, b3, gamma, beta, w1, b1):
    N, Cin, H, W = x_nchw.shape
    Cout = w3.shape[-1]
    HW = H * W
    P = N * HW
    g = math.gcd(GIMG, N)
    ng = N // g
    pb = math.gcd(PB, HW)
    np_ = HW // pb

    cparams = pltpu.CompilerParams(
        dimension_semantics=("parallel",),
        vmem_limit_bytes=64 * 1024 * 1024,
    )

    xv = jnp.transpose(x_nchw, (1, 2, 3, 0)).reshape(Cin, HW, N)
    xv = xv.astype(jnp.float32)
    tb = min(128, N)
    ntb = N // tb
    cparams_t = pltpu.CompilerParams(
        dimension_semantics=("parallel", "parallel"),
        vmem_limit_bytes=64 * 1024 * 1024,
    )
    xt = pl.pallas_call(
        _tin_kernel,
        grid=(ntb, np_),
        in_specs=[pl.BlockSpec((Cin, pb, tb), lambda b, j: (0, j, b))],
        out_specs=pl.BlockSpec((tb, Cin, pb), lambda b, j: (b, 0, j)),
        out_shape=jax.ShapeDtypeStruct((N, Cin, HW), jnp.bfloat16),
        compiler_params=cparams_t,
        cost_estimate=pl.CostEstimate(
            flops=0, transcendentals=0,
            bytes_accessed=int(4 * Cin * HW * N + 2 * Cin * HW * N)),
    )(xv)
    x = xt.reshape(ng, g * Cin, HW)

    w9 = jnp.transpose(w3.astype(jnp.float32),
                       (0, 1, 3, 2)).reshape(9, Cout, Cin).astype(jnp.bfloat16)
    w1t = jnp.transpose(w1.astype(jnp.float32)).astype(jnp.bfloat16)
    b3b = jnp.broadcast_to(b3.reshape(Cout, 1).astype(jnp.float32),
                           (Cout, 128))

    flops1 = int(N * 9 * Cout * Cin * HW * 2 + N * 6 * Cout * HW)
    bytes1 = int(2 * N * Cin * HW + 2 * N * Cout * HW + 2 * 9 * Cout * Cin
                 + 4 * (Cout * 128 + ng * Cout * 2))
    y1, stats = pl.pallas_call(
        partial(_p1_kernel, G=g, W=W, Cin=Cin, Cout=Cout),
        grid=(ng,),
        in_specs=[
            pl.BlockSpec((1, g * Cin, HW), lambda n: (n, 0, 0)),
            pl.BlockSpec((9, Cout, Cin), lambda n: (0, 0, 0)),
            pl.BlockSpec((Cout, 128), lambda n: (0, 0)),
        ],
        out_specs=(
            pl.BlockSpec((1, g * Cout, HW), lambda n: (n, 0, 0)),
            pl.BlockSpec((1, Cout, 2), lambda n: (n, 0, 0)),
        ),
        out_shape=(
            jax.ShapeDtypeStruct((ng, g * Cout, HW), jnp.bfloat16),
            jax.ShapeDtypeStruct((ng, Cout, 2), jnp.float32),
        ),
        compiler_params=cparams,
        cost_estimate=pl.CostEstimate(flops=flops1, transcendentals=0,
                                      bytes_accessed=bytes1),
    )(x, w9, b3b)

    s = stats.sum(axis=0)
    mean = s[:, 0] / P
    var = s[:, 1] / P - mean * mean
    scale = gamma.reshape(Cout) * lax.rsqrt(var + EPS)
    shift = beta.reshape(Cout) - mean * scale
    ssb = jnp.broadcast_to(
        jnp.stack([scale, shift, b1.reshape(Cout).astype(jnp.float32)]
                  )[:, :, None], (3, Cout, 128))

    flops2 = int(N * Cout * Cin * HW * 2 + N * 6 * Cout * HW)
    bytes2 = int(4 * N * Cin * HW + 2 * N * Cout * HW + 2 * Cout * Cin
                 + 4 * 3 * Cout * 128 + 4 * N * Cout * HW)
    nb = min(128, N)
    nnb = N // nb
    cparams2 = pltpu.CompilerParams(
        dimension_semantics=("parallel", "parallel"),
        vmem_limit_bytes=64 * 1024 * 1024,
    )
    oc = pl.pallas_call(
        partial(_p2_kernel, Cin=Cin, Cout=Cout, N=nb),
        grid=(nnb, np_),
        in_specs=[
            pl.BlockSpec((Cin, pb, nb), lambda b, j: (0, j, b)),
            pl.BlockSpec((nb, Cout, pb), lambda b, j: (b, 0, j)),
            pl.BlockSpec((Cout, Cin), lambda b, j: (0, 0)),
            pl.BlockSpec((3, Cout, 128), lambda b, j: (0, 0, 0)),
        ],
        out_specs=pl.BlockSpec((Cout, pb, nb), lambda b, j: (0, j, b)),
        out_shape=jax.ShapeDtypeStruct((Cout, HW, N), jnp.float32),
        compiler_params=cparams2,
        cost_estimate=pl.CostEstimate(flops=flops2, transcendentals=0,
                                      bytes_accessed=bytes2),
    )(xv, y1.reshape(N, Cout, HW), w1t, ssb)

    return jnp.transpose(oc.reshape(Cout, H, W, N), (3, 0, 1, 2))


def kernel(x_nchw, w3, b3, gamma, beta, w1, b1):
    return _forward(x_nchw, w3, b3, gamma, beta, w1, b1)


# speedup vs baseline: 1.0271x; 1.0271x over previous
import math
from functools import partial

import jax
import jax.numpy as jnp
from jax import lax
from jax.experimental import pallas as pl
from jax.experimental.pallas import tpu as pltpu

EPS = 1e-5
GIMG = 32
PB = 128


def _shift_lanes(x, s, zcol):
    if s == 0:
        return x
    if s > 0:
        return jnp.concatenate([x[:, s:], zcol[:, :s]], axis=1)
    return jnp.concatenate([zcol[:, :(-s)], x[:, :s]], axis=1)


def _tin_kernel(x_ref, o_ref):
    o_ref[...] = jnp.transpose(x_ref[...], (2, 0, 1)).astype(jnp.bfloat16)


def _p1_kernel(x_ref, w_ref, b3_ref, y1_ref, st_ref, *, G, W, Cin, Cout):
    xb = x_ref[0]
    rows, hw = xb.shape
    zcol = jnp.zeros((rows, 33), jnp.bfloat16)
    lane = lax.broadcasted_iota(jnp.int32, (1, hw), 1) % W
    zero = jnp.zeros((), jnp.bfloat16)
    shifted = []
    for ky in range(3):
        for kx in range(3):
            s = W * (ky - 1) + (kx - 1)
            t = _shift_lanes(xb, s, zcol)
            if kx == 0:
                t = jnp.where(lane == 0, zero, t)
            elif kx == 2:
                t = jnp.where(lane == W - 1, zero, t)
            shifted.append(t)
    b3c = b3_ref[:, 0:1]
    for i in range(G):
        r0 = i * Cin
        acc = jnp.dot(w_ref[0], shifted[0][r0:r0 + Cin, :],
                      preferred_element_type=jnp.float32)
        for k in range(1, 9):
            acc = acc + jnp.dot(w_ref[k], shifted[k][r0:r0 + Cin, :],
                                preferred_element_type=jnp.float32)
        y = acc + b3c
        y1_ref[0, i * Cout:(i + 1) * Cout, :] = y.astype(jnp.bfloat16)
        s1 = jnp.sum(y, axis=1, keepdims=True)
        s2 = jnp.sum(y * y, axis=1, keepdims=True)
        if i == 0:
            st1, st2 = s1, s2
        else:
            st1, st2 = st1 + s1, st2 + s2
    st_ref[0] = jnp.concatenate([st1, st2], axis=1)


def _p2_kernel(xv_ref, y1_ref, w1_ref, ss_ref, o_ref, *, Cin, Cout, N):
    pb = xv_ref.shape[1]
    xb = xv_ref[...].reshape(Cin, pb * N).astype(jnp.bfloat16)
    y2 = jnp.dot(w1_ref[...], xb,
                 preferred_element_type=jnp.float32).reshape(Cout, pb, N)
    y1c = jnp.transpose(y1_ref[...], (1, 2, 0)).astype(jnp.float32)
    ss = ss_ref[...]
    sc = ss[0][:, 0:1, None]
    sh = ss[1][:, 0:1, None]
    b1c = ss[2][:, 0:1, None]
    y1n = jnp.maximum(y1c * sc + sh, 0.0)
    o_ref[...] = jnp.maximum(y1n + y2 + b1c, 0.0)


@jax.jit
def _forward(x_nchw, w3, b3, gamma, beta, w1, b1):
    N, Cin, H, W = x_nchw.shape
    Cout = w3.shape[-1]
    HW = H * W
    P = N * HW
    g = math.gcd(GIMG, N)
    ng = N // g
    pb = math.gcd(PB, HW)
    np_ = HW // pb

    cparams = pltpu.CompilerParams(
        dimension_semantics=("parallel",),
        vmem_limit_bytes=64 * 1024 * 1024,
    )

    xv = jnp.transpose(x_nchw, (1, 2, 3, 0)).reshape(Cin, HW, N)
    xv = xv.astype(jnp.float32)
    xt = pl.pallas_call(
        _tin_kernel,
        grid=(np_,),
        in_specs=[pl.BlockSpec((Cin, pb, N), lambda j: (0, j, 0))],
        out_specs=pl.BlockSpec((N, Cin, pb), lambda j: (0, 0, j)),
        out_shape=jax.ShapeDtypeStruct((N, Cin, HW), jnp.bfloat16),
        compiler_params=cparams,
        cost_estimate=pl.CostEstimate(
            flops=0, transcendentals=0,
            bytes_accessed=int(4 * Cin * HW * N + 2 * Cin * HW * N)),
    )(xv)
    x = xt.reshape(ng, g * Cin, HW)

    w9 = jnp.transpose(w3.astype(jnp.float32),
                       (0, 1, 3, 2)).reshape(9, Cout, Cin).astype(jnp.bfloat16)
    w1t = jnp.transpose(w1.astype(jnp.float32)).astype(jnp.bfloat16)
    b3b = jnp.broadcast_to(b3.reshape(Cout, 1).astype(jnp.float32),
                           (Cout, 128))

    flops1 = int(N * 9 * Cout * Cin * HW * 2 + N * 6 * Cout * HW)
    bytes1 = int(2 * N * Cin * HW + 2 * N * Cout * HW + 2 * 9 * Cout * Cin
                 + 4 * (Cout * 128 + ng * Cout * 2))
    y1, stats = pl.pallas_call(
        partial(_p1_kernel, G=g, W=W, Cin=Cin, Cout=Cout),
        grid=(ng,),
        in_specs=[
            pl.BlockSpec((1, g * Cin, HW), lambda n: (n, 0, 0)),
            pl.BlockSpec((9, Cout, Cin), lambda n: (0, 0, 0)),
            pl.BlockSpec((Cout, 128), lambda n: (0, 0)),
        ],
        out_specs=(
            pl.BlockSpec((1, g * Cout, HW), lambda n: (n, 0, 0)),
            pl.BlockSpec((1, Cout, 2), lambda n: (n, 0, 0)),
        ),
        out_shape=(
            jax.ShapeDtypeStruct((ng, g * Cout, HW), jnp.bfloat16),
            jax.ShapeDtypeStruct((ng, Cout, 2), jnp.float32),
        ),
        compiler_params=cparams,
        cost_estimate=pl.CostEstimate(flops=flops1, transcendentals=0,
                                      bytes_accessed=bytes1),
    )(x, w9, b3b)

    s = stats.sum(axis=0)
    mean = s[:, 0] / P
    var = s[:, 1] / P - mean * mean
    scale = gamma.reshape(Cout) * lax.rsqrt(var + EPS)
    shift = beta.reshape(Cout) - mean * scale
    ssb = jnp.broadcast_to(
        jnp.stack([scale, shift, b1.reshape(Cout).astype(jnp.float32)]
                  )[:, :, None], (3, Cout, 128))

    flops2 = int(N * Cout * Cin * HW * 2 + N * 6 * Cout * HW)
    bytes2 = int(4 * N * Cin * HW + 2 * N * Cout * HW + 2 * Cout * Cin
                 + 4 * 3 * Cout * 128 + 4 * N * Cout * HW)
    nb = min(128, N)
    nnb = N // nb
    cparams2 = pltpu.CompilerParams(
        dimension_semantics=("parallel", "parallel"),
        vmem_limit_bytes=64 * 1024 * 1024,
    )
    oc = pl.pallas_call(
        partial(_p2_kernel, Cin=Cin, Cout=Cout, N=nb),
        grid=(nnb, np_),
        in_specs=[
            pl.BlockSpec((Cin, pb, nb), lambda b, j: (0, j, b)),
            pl.BlockSpec((nb, Cout, pb), lambda b, j: (b, 0, j)),
            pl.BlockSpec((Cout, Cin), lambda b, j: (0, 0)),
            pl.BlockSpec((3, Cout, 128), lambda b, j: (0, 0, 0)),
        ],
        out_specs=pl.BlockSpec((Cout, pb, nb), lambda b, j: (0, j, b)),
        out_shape=jax.ShapeDtypeStruct((Cout, HW, N), jnp.float32),
        compiler_params=cparams2,
        cost_estimate=pl.CostEstimate(flops=flops2, transcendentals=0,
                                      bytes_accessed=bytes2),
    )(xv, y1.reshape(N, Cout, HW), w1t, ssb)

    return jnp.transpose(oc.reshape(Cout, H, W, N), (3, 0, 1, 2))


def kernel(x_nchw, w3, b3, gamma, beta, w1, b1):
    return _forward(x_nchw, w3, b3, gamma, beta, w1, b1)
